# Initial kernel scaffold; baseline (speedup 1.0000x reference)
import functools

import jax
import jax.numpy as jnp
from jax import lax
from jax.experimental import pallas as pl
from jax.experimental.pallas import tpu as pltpu
from jax.experimental.pallas import tpu_sc as plsc

NUM_CORES = 2
NUM_SUBCORES = 16
NUM_WORKERS = NUM_CORES * NUM_SUBCORES


@functools.partial(jax.jit, static_argnums=(2, 3))
def _sc_gather(idx, weight, chunk, nchunk):
    total = idx.shape[0]
    dim = weight.shape[1]
    b_per_w = total // NUM_WORKERS
    mesh = plsc.VectorSubcoreMesh(core_axis_name="c", subcore_axis_name="s")

    @functools.partial(
        pl.kernel,
        mesh=mesh,
        out_type=jax.ShapeDtypeStruct((total, dim), jnp.float32),
        scratch_types=[
            pltpu.VMEM((b_per_w,), jnp.int32),
            pltpu.VMEM((chunk, dim), jnp.float32),
            pltpu.SemaphoreType.DMA,
        ],
    )
    def body(idx_hbm, table_hbm, out_hbm, idx_v, rows_v, sem):
        wid = lax.axis_index("s") * NUM_CORES + lax.axis_index("c")
        base = wid * b_per_w
        pltpu.sync_copy(idx_hbm.at[pl.ds(base, b_per_w)], idx_v)
        for c in range(nchunk):
            pltpu.async_copy(
                table_hbm.at[idx_v.at[pl.ds(c * chunk, chunk)]], rows_v, sem
            ).wait()
            pltpu.sync_copy(rows_v, out_hbm.at[pl.ds(base + c * chunk, chunk)])

    return body(idx, weight)


def kernel(x, weight):
    batch, fields = x.shape
    dim = weight.shape[1]
    total = batch * fields
    flat = x.reshape(total).astype(jnp.int32)
    b_per_w = total // NUM_WORKERS
    nchunk = 8
    chunk = b_per_w // nchunk
    out = _sc_gather(flat, weight, chunk, nchunk)
    return out.reshape(batch, fields, dim)


# speedup vs baseline: 1.5692x; 1.5692x over previous
import functools

import jax
import jax.numpy as jnp
from jax import lax
from jax.experimental import pallas as pl
from jax.experimental.pallas import tpu as pltpu
from jax.experimental.pallas import tpu_sc as plsc

NUM_CORES = 2
NUM_SUBCORES = 16
NUM_WORKERS = NUM_CORES * NUM_SUBCORES


@functools.partial(jax.jit, static_argnums=(2, 3))
def _sc_gather(idx, weight, chunk, nchunk):
    total = idx.shape[0]
    dim = weight.shape[1]
    b_per_w = total // NUM_WORKERS
    mesh = plsc.VectorSubcoreMesh(core_axis_name="c", subcore_axis_name="s")

    @functools.partial(
        pl.kernel,
        mesh=mesh,
        out_type=jax.ShapeDtypeStruct((total, dim), jnp.float32),
        scratch_types=[
            pltpu.VMEM((b_per_w,), jnp.int32),
            pltpu.VMEM((chunk, dim), jnp.float32),
            pltpu.SemaphoreType.DMA,
        ],
        compiler_params=pltpu.CompilerParams(use_tc_tiling_on_sc=False),
    )
    def body(idx_hbm, table_hbm, out_hbm, idx_v, rows_v, sem):
        wid = lax.axis_index("s") * NUM_CORES + lax.axis_index("c")
        base = wid * b_per_w
        pltpu.sync_copy(idx_hbm.at[pl.ds(base, b_per_w)], idx_v)
        for c in range(nchunk):
            pltpu.async_copy(
                table_hbm.at[idx_v.at[pl.ds(c * chunk, chunk)]], rows_v, sem
            ).wait()
            pltpu.sync_copy(rows_v, out_hbm.at[pl.ds(base + c * chunk, chunk)])

    return body(idx, weight)


def kernel(x, weight):
    batch, fields = x.shape
    dim = weight.shape[1]
    total = batch * fields
    flat = x.reshape(total).astype(jnp.int32)
    b_per_w = total // NUM_WORKERS
    nchunk = 8
    chunk = b_per_w // nchunk
    out = _sc_gather(flat, weight, chunk, nchunk)
    return out.reshape(batch, fields, dim)


# speedup vs baseline: 1.5755x; 1.0040x over previous
import functools

import jax
import jax.numpy as jnp
from jax import lax
from jax.experimental import pallas as pl
from jax.experimental.pallas import tpu as pltpu
from jax.experimental.pallas import tpu_sc as plsc

NUM_CORES = 2
NUM_SUBCORES = 16
NUM_WORKERS = NUM_CORES * NUM_SUBCORES


NBUF = 4


@functools.partial(jax.jit, static_argnums=(2, 3))
def _sc_gather(idx, weight, chunk, nchunk):
    total = idx.shape[0]
    dim = weight.shape[1]
    b_per_w = total // NUM_WORKERS
    mesh = plsc.VectorSubcoreMesh(core_axis_name="c", subcore_axis_name="s")

    @functools.partial(
        pl.kernel,
        mesh=mesh,
        out_type=jax.ShapeDtypeStruct((total, dim), jnp.float32),
        scratch_types=[
            pltpu.VMEM((b_per_w,), jnp.int32),
            [pltpu.VMEM((chunk, dim), jnp.float32) for _ in range(NBUF)],
            [pltpu.SemaphoreType.DMA for _ in range(NBUF)],
            [pltpu.SemaphoreType.DMA for _ in range(NBUF)],
        ],
        compiler_params=pltpu.CompilerParams(use_tc_tiling_on_sc=False),
    )
    def body(idx_hbm, table_hbm, out_hbm, idx_v, rows_v, gsem, ssem):
        wid = lax.axis_index("s") * NUM_CORES + lax.axis_index("c")
        base = wid * b_per_w

        def gather(c, b):
            return pltpu.async_copy(
                table_hbm.at[idx_v.at[pl.ds(c * chunk, chunk)]], rows_v[b], gsem[b]
            )

        def store(c, b):
            return pltpu.async_copy(
                rows_v[b], out_hbm.at[pl.ds(base + c * chunk, chunk)], ssem[b]
            )

        pltpu.sync_copy(idx_hbm.at[pl.ds(base, b_per_w)], idx_v)
        g = [None] * NBUF
        for b in range(min(NBUF, nchunk)):
            g[b] = gather(b, b)
        for c in range(nchunk):
            b = c % NBUF
            g[b].wait()
            s = store(c, b)
            nxt = c + NBUF
            if nxt < nchunk:
                s.wait()
                g[b] = gather(nxt, b)
            else:
                s.wait()

    return body(idx, weight)


def kernel(x, weight):
    batch, fields = x.shape
    dim = weight.shape[1]
    total = batch * fields
    flat = x.reshape(total).astype(jnp.int32)
    b_per_w = total // NUM_WORKERS
    nchunk = 16
    chunk = b_per_w // nchunk
    out = _sc_gather(flat, weight, chunk, nchunk)
    return out.reshape(batch, fields, dim)
